# Initial kernel scaffold; baseline (speedup 1.0000x reference)
import jax
import jax.numpy as jnp
from jax.experimental import pallas as pl


def kernel(regimes, table):
    raise NotImplementedError("write your pallas kernel here")


# speedup vs baseline: 4.4314x; 4.4314x over previous
import functools

import jax
import jax.numpy as jnp
from jax import lax
from jax.experimental import pallas as pl
from jax.experimental.pallas import tpu as pltpu
from jax.experimental.pallas import tpu_sc as plsc

NUM_CORES = 2
NUM_SUBCORES = 16
NUM_WORKERS = NUM_CORES * NUM_SUBCORES
EMBED = 32
CHUNK = 128


def _body(table_hbm, idx_hbm, out_hbm, idx_v, rows_v, sem):
    wid = lax.axis_index("s") * NUM_CORES + lax.axis_index("c")
    n_per_w = idx_hbm.shape[0] // NUM_WORKERS
    nchunk = n_per_w // CHUNK
    base = wid * n_per_w

    def step(j, carry):
        off = base + j * CHUNK
        pltpu.sync_copy(idx_hbm.at[pl.ds(off, CHUNK)], idx_v)
        pltpu.async_copy(table_hbm.at[idx_v], rows_v, sem).wait()
        pltpu.sync_copy(rows_v, out_hbm.at[pl.ds(off, CHUNK)])
        return carry

    lax.fori_loop(0, nchunk, step, 0)


@functools.partial(jax.jit, static_argnames=("n",))
def _gather(table, idx, n):
    mesh = plsc.VectorSubcoreMesh(
        core_axis_name="c", subcore_axis_name="s",
        num_cores=NUM_CORES, num_subcores=NUM_SUBCORES)
    return pl.kernel(
        _body,
        out_type=jax.ShapeDtypeStruct((n, EMBED), jnp.float32),
        mesh=mesh,
        scratch_types=[
            pltpu.VMEM((CHUNK,), jnp.int32),
            pltpu.VMEM((CHUNK, EMBED), jnp.float32),
            pltpu.SemaphoreType.DMA,
        ],
        compiler_params=pltpu.CompilerParams(use_tc_tiling_on_sc=False),
    )(table, idx)


def kernel(regimes, table):
    b, t = regimes.shape
    idx = regimes.reshape(-1).astype(jnp.int32)
    out = _gather(table, idx, idx.shape[0])
    return out.reshape(b, t, EMBED)


# speedup vs baseline: 5.7584x; 1.2995x over previous
import functools

import jax
import jax.numpy as jnp
from jax import lax
from jax.experimental import pallas as pl
from jax.experimental.pallas import tpu as pltpu
from jax.experimental.pallas import tpu_sc as plsc

NUM_CORES = 2
NUM_SUBCORES = 16
NUM_WORKERS = NUM_CORES * NUM_SUBCORES
EMBED = 32
CHUNK = 512


def _body(table_hbm, idx_hbm, out_hbm, idx_v, rows_v, sem):
    wid = lax.axis_index("s") * NUM_CORES + lax.axis_index("c")
    n_per_w = idx_hbm.shape[0] // NUM_WORKERS
    nchunk = n_per_w // CHUNK
    base = wid * n_per_w

    def step(j, carry):
        off = base + j * CHUNK
        pltpu.sync_copy(idx_hbm.at[pl.ds(off, CHUNK)], idx_v)
        pltpu.async_copy(table_hbm.at[idx_v], rows_v, sem).wait()
        pltpu.sync_copy(rows_v, out_hbm.at[pl.ds(off, CHUNK)])
        return carry

    lax.fori_loop(0, nchunk, step, 0)


@functools.partial(jax.jit, static_argnames=("n",))
def _gather(table, idx, n):
    mesh = plsc.VectorSubcoreMesh(
        core_axis_name="c", subcore_axis_name="s",
        num_cores=NUM_CORES, num_subcores=NUM_SUBCORES)
    return pl.kernel(
        _body,
        out_type=jax.ShapeDtypeStruct((n, EMBED), jnp.float32),
        mesh=mesh,
        scratch_types=[
            pltpu.VMEM((CHUNK,), jnp.int32),
            pltpu.VMEM((CHUNK, EMBED), jnp.float32),
            pltpu.SemaphoreType.DMA,
        ],
        compiler_params=pltpu.CompilerParams(use_tc_tiling_on_sc=False),
    )(table, idx)


def kernel(regimes, table):
    b, t = regimes.shape
    idx = regimes.reshape(-1).astype(jnp.int32)
    out = _gather(table, idx, idx.shape[0])
    return out.reshape(b, t, EMBED)


# speedup vs baseline: 6.4661x; 1.1229x over previous
import functools

import jax
import jax.numpy as jnp
from jax import lax
from jax.experimental import pallas as pl
from jax.experimental.pallas import tpu as pltpu
from jax.experimental.pallas import tpu_sc as plsc

NUM_CORES = 2
NUM_SUBCORES = 16
NUM_WORKERS = NUM_CORES * NUM_SUBCORES
EMBED = 32
CHUNK = 512


def _body(table_hbm, idx_hbm, out_hbm, idx_v, rows_v,
          sem_i0, sem_i1, sem_g0, sem_g1, sem_o0, sem_o1):
    wid = lax.axis_index("s") * NUM_CORES + lax.axis_index("c")
    n_per_w = idx_hbm.shape[0] // NUM_WORKERS
    nchunk = n_per_w // CHUNK
    base = wid * n_per_w
    sem_i = (sem_i0, sem_i1)
    sem_g = (sem_g0, sem_g1)
    sem_o = (sem_o0, sem_o1)

    def idx_copy(j, b):
        return pltpu.make_async_copy(
            idx_hbm.at[pl.ds(base + j * CHUNK, CHUNK)], idx_v.at[b], sem_i[b])

    def gather_copy(b):
        return pltpu.make_async_copy(table_hbm.at[idx_v.at[b]], rows_v.at[b],
                                     sem_g[b])

    def out_copy(j, b):
        return pltpu.make_async_copy(
            rows_v.at[b], out_hbm.at[pl.ds(base + j * CHUNK, CHUNK)], sem_o[b])

    idx_copy(0, 0).start()
    idx_copy(1, 1).start()
    idx_copy(0, 0).wait()
    gather_copy(0).start()

    def step(jo, carry):
        for b in range(2):
            j = jo * 2 + b
            o = 1 - b

            @pl.when(j >= 1)
            def _():
                out_copy(j - 1, o).wait()

            @pl.when(j + 1 < nchunk)
            def _():
                idx_copy(j + 1, o).wait()
                gather_copy(o).start()

            gather_copy(b).wait()

            @pl.when(j + 2 < nchunk)
            def _():
                idx_copy(j + 2, b).start()

            out_copy(j, b).start()
        return carry

    lax.fori_loop(0, nchunk // 2, step, 0)
    out_copy(nchunk - 1, (nchunk - 1) % 2).wait()


@functools.partial(jax.jit, static_argnames=("n",))
def _gather(table, idx, n):
    mesh = plsc.VectorSubcoreMesh(
        core_axis_name="c", subcore_axis_name="s",
        num_cores=NUM_CORES, num_subcores=NUM_SUBCORES)
    return pl.kernel(
        _body,
        out_type=jax.ShapeDtypeStruct((n, EMBED), jnp.float32),
        mesh=mesh,
        scratch_types=[
            pltpu.VMEM((2, CHUNK), jnp.int32),
            pltpu.VMEM((2, CHUNK, EMBED), jnp.float32),
            pltpu.SemaphoreType.DMA,
            pltpu.SemaphoreType.DMA,
            pltpu.SemaphoreType.DMA,
            pltpu.SemaphoreType.DMA,
            pltpu.SemaphoreType.DMA,
            pltpu.SemaphoreType.DMA,
        ],
        compiler_params=pltpu.CompilerParams(use_tc_tiling_on_sc=False),
    )(table, idx)


def kernel(regimes, table):
    b, t = regimes.shape
    idx = regimes.reshape(-1).astype(jnp.int32)
    out = _gather(table, idx, idx.shape[0])
    return out.reshape(b, t, EMBED)


# speedup vs baseline: 6.5003x; 1.0053x over previous
import functools

import jax
import jax.numpy as jnp
from jax import lax
from jax.experimental import pallas as pl
from jax.experimental.pallas import tpu as pltpu
from jax.experimental.pallas import tpu_sc as plsc

NUM_CORES = 2
NUM_SUBCORES = 16
NUM_WORKERS = NUM_CORES * NUM_SUBCORES
EMBED = 32
CHUNK = 512
NBUF = 4


def _body(table_hbm, idx_hbm, out_hbm, idx_v, rows_v, sem_i, sem_g, sem_o):
    wid = lax.axis_index("s") * NUM_CORES + lax.axis_index("c")
    n_per_w = idx_hbm.shape[0] // NUM_WORKERS
    nchunk = n_per_w // CHUNK
    base = wid * n_per_w

    def idx_copy(j, b):
        return pltpu.make_async_copy(
            idx_hbm.at[pl.ds(base + j * CHUNK, CHUNK)], idx_v.at[b],
            sem_i.at[b])

    def gather_copy(b):
        return pltpu.make_async_copy(table_hbm.at[idx_v.at[b]], rows_v.at[b],
                                     sem_g.at[b])

    def out_copy(j, b):
        return pltpu.make_async_copy(
            rows_v.at[b], out_hbm.at[pl.ds(base + j * CHUNK, CHUNK)],
            sem_o.at[b])

    for k in range(NBUF):
        idx_copy(k, k).start()
    for k in range(2):
        idx_copy(k, k).wait()
        gather_copy(k).start()

    def step(jo, carry):
        for b in range(NBUF):
            j = jo * NBUF + b
            bg = (b + 2) % NBUF

            @pl.when(j + 2 < nchunk)
            def _():
                @pl.when(j >= 2)
                def _():
                    out_copy(j - 2, bg).wait()
                idx_copy(j + 2, bg).wait()
                gather_copy(bg).start()

            gather_copy(b).wait()

            @pl.when(j + NBUF < nchunk)
            def _():
                idx_copy(j + NBUF, b).start()

            out_copy(j, b).start()
        return carry

    lax.fori_loop(0, nchunk // NBUF, step, 0)
    for k in range(nchunk - 4, nchunk):
        out_copy(k, k % NBUF).wait()


@functools.partial(jax.jit, static_argnames=("n",))
def _gather(table, idx, n):
    mesh = plsc.VectorSubcoreMesh(
        core_axis_name="c", subcore_axis_name="s",
        num_cores=NUM_CORES, num_subcores=NUM_SUBCORES)
    return pl.kernel(
        _body,
        out_type=jax.ShapeDtypeStruct((n, EMBED), jnp.float32),
        mesh=mesh,
        scratch_types=[
            pltpu.VMEM((NBUF, CHUNK), jnp.int32),
            pltpu.VMEM((NBUF, CHUNK, EMBED), jnp.float32),
            pltpu.SemaphoreType.DMA((NBUF,)),
            pltpu.SemaphoreType.DMA((NBUF,)),
            pltpu.SemaphoreType.DMA((NBUF,)),
        ],
        compiler_params=pltpu.CompilerParams(use_tc_tiling_on_sc=False),
    )(table, idx)


def kernel(regimes, table):
    b, t = regimes.shape
    idx = regimes.reshape(-1).astype(jnp.int32)
    out = _gather(table, idx, idx.shape[0])
    return out.reshape(b, t, EMBED)
